# baseline (speedup 1.0000x reference)
import dataclasses
import functools

import jax
import jax.numpy as jnp
from jax import lax
from jax.experimental import pallas as pl
from jax.experimental.pallas import tpu as pltpu
from jax.experimental.pallas import tpu_sc as plsc

B = 16
N = 4096
C = 128
H = 64
NC = 256
ROWS = B * N
BLK = 512
SC_WORKERS = 32
NGROUPS = 2
BG = B // NGROUPS
GROWS = ROWS // NGROUPS
NACC = 4
WPB = SC_WORKERS * NACC // BG
ROWS_PER_W = GROWS // SC_WORKERS
SC_CHUNK = 256
F32 = jnp.float32
I32 = jnp.int32

HIGHEST = lax.Precision.HIGHEST


def _ln_relu(h, g, beta):
    mu = jnp.mean(h, axis=-1, keepdims=True)
    var = jnp.mean((h - mu) ** 2, axis=-1, keepdims=True)
    h = (h - mu) * lax.rsqrt(var + 1e-5) * g + beta
    return jnp.maximum(h, 0.0)


def _bf16_dot(a, b):
    return lax.dot_general(a.astype(jnp.bfloat16), b.astype(jnp.bfloat16),
                           (((1,), (0,)), ((), ())),
                           preferred_element_type=F32)


def _l0_body(x_ref, w_ref, b_ref, g_ref, beta_ref, o_ref):
    h = _bf16_dot(x_ref[...], w_ref[...])
    o_ref[...] = _ln_relu(h + b_ref[...], g_ref[...], beta_ref[...])


def _layer0(x2d, W0, b0, g0, beta0):
    return pl.pallas_call(
        _l0_body,
        grid=(ROWS // BLK,),
        in_specs=[
            pl.BlockSpec((BLK, C), lambda i: (i, 0)),
            pl.BlockSpec((C, H), lambda i: (0, 0)),
            pl.BlockSpec((1, H), lambda i: (0, 0)),
            pl.BlockSpec((1, H), lambda i: (0, 0)),
            pl.BlockSpec((1, H), lambda i: (0, 0)),
        ],
        out_specs=pl.BlockSpec((BLK, H), lambda i: (i, 0)),
        out_shape=jax.ShapeDtypeStruct((ROWS, H), F32),
    )(x2d, W0, b0.reshape(1, H), g0.reshape(1, H), beta0.reshape(1, H))


def _merge_partials(p):
    m = p[0]
    for i in range(1, p.shape[0]):
        m = jnp.maximum(m, p[i])
    return m


def _tmerge_body(p_ref, wb_ref, o_ref):
    aggr = _merge_partials(p_ref[0])
    o_ref[0] = _bf16_dot(aggr, wb_ref[...])


def _tmerge(partials, W_bot):
    return pl.pallas_call(
        _tmerge_body,
        grid=(BG,),
        in_specs=[
            pl.BlockSpec((1, WPB, NC, H), lambda b_: (b_, 0, 0, 0)),
            pl.BlockSpec((H, H), lambda b_: (0, 0)),
        ],
        out_specs=pl.BlockSpec((1, NC, H), lambda b_: (b_, 0, 0)),
        out_shape=jax.ShapeDtypeStruct((BG, NC, H), F32),
    )(partials, W_bot)


def _lmid_body(h_ref, cl_ref, t_ref, wt_ref, b_ref, g_ref, beta_ref,
               o_ref):
    t = t_ref[0]
    cl = cl_ref[0, 0, :]
    bf = jnp.bfloat16
    oh = jnp.where(
        lax.broadcasted_iota(I32, (BLK, NC), 1) == cl[:, None],
        1.0, 0.0).astype(bf)
    t_hi = t.astype(bf)
    t_lo = (t - t_hi.astype(F32)).astype(bf)
    dims = (((1,), (0,)), ((), ()))
    contrib = (lax.dot_general(oh, t_hi, dims, preferred_element_type=F32) +
               lax.dot_general(oh, t_lo, dims, preferred_element_type=F32))
    h = _bf16_dot(h_ref[...], wt_ref[...])
    h = h + contrib + b_ref[...]
    o_ref[...] = _ln_relu(h, g_ref[...], beta_ref[...])


def _layer_mid(h2d, cl3d, tbl, W, b, g, beta):
    blocks_per_b = N // BLK
    return pl.pallas_call(
        _lmid_body,
        grid=(BG, blocks_per_b),
        in_specs=[
            pl.BlockSpec((BLK, H), lambda b_, i: (b_ * blocks_per_b + i, 0)),
            pl.BlockSpec((1, 1, BLK),
                         lambda b_, i: (b_ * blocks_per_b + i, 0, 0)),
            pl.BlockSpec((1, NC, H), lambda b_, i: (b_, 0, 0)),
            pl.BlockSpec((H, H), lambda b_, i: (0, 0)),
            pl.BlockSpec((1, H), lambda b_, i: (0, 0)),
            pl.BlockSpec((1, H), lambda b_, i: (0, 0)),
            pl.BlockSpec((1, H), lambda b_, i: (0, 0)),
        ],
        out_specs=pl.BlockSpec((BLK, H),
                               lambda b_, i: (b_ * blocks_per_b + i, 0)),
        out_shape=jax.ShapeDtypeStruct((GROWS, H), F32),
    )(h2d, cl3d, tbl, W[:H], b.reshape(1, H), g.reshape(1, H),
      beta.reshape(1, H))


def _final_body(p_ref, o_ref):
    aggr = _merge_partials(p_ref[0])
    norm = jnp.sqrt(jnp.sum(aggr * aggr, axis=0, keepdims=True))
    normed = aggr / jnp.maximum(norm, 1e-12)
    o_ref[0] = jnp.concatenate([normed, normed], axis=-1)


def _final(partials):
    return pl.pallas_call(
        _final_body,
        grid=(B,),
        in_specs=[pl.BlockSpec((1, WPB, NC, H), lambda b_: (b_, 0, 0, 0))],
        out_specs=pl.BlockSpec((1, NC, 2 * H), lambda b_: (b_, 0, 0)),
        out_shape=jax.ShapeDtypeStruct((B, NC, 2 * H), F32),
    )(partials)


def _sc_compiler_params():
    cp = pltpu.CompilerParams()
    if "needs_layout_passes" in pltpu.CompilerParams.__dataclass_fields__:
        cp = dataclasses.replace(cp, needs_layout_passes=False)
    return cp


def _sc_scatter_max(h_flat, cl_flat):
    mesh = plsc.VectorSubcoreMesh(core_axis_name="c", subcore_axis_name="s")

    @functools.partial(
        pl.kernel,
        out_type=jax.ShapeDtypeStruct((SC_WORKERS * NACC, NC * H), F32),
        mesh=mesh,
        compiler_params=_sc_compiler_params(),
        scratch_types=[
            [pltpu.VMEM((NC * H,), F32) for _ in range(NACC)],
            pltpu.VMEM((SC_CHUNK * H,), F32),
            pltpu.VMEM((SC_CHUNK * H,), F32),
            pltpu.VMEM((SC_CHUNK,), I32),
            pltpu.VMEM((SC_CHUNK,), I32),
            pltpu.SemaphoreType.DMA,
            pltpu.SemaphoreType.DMA,
            pltpu.SemaphoreType.DMA,
            pltpu.SemaphoreType.DMA,
            pltpu.SemaphoreType.DMA,
        ],
    )
    def sc_kernel(h_hbm, cl_hbm, z_hbm, out_hbm, accs, hb0, hb1, cb0, cb1,
                  sh0, sh1, sc0, sc1, sz):
        w = lax.axis_index("c") * 16 + lax.axis_index("s")
        base = w * ROWS_PER_W

        zcopies = [pltpu.make_async_copy(z_hbm, a, sz) for a in accs]
        for zc in zcopies:
            zc.start()

        hbufs, cbufs = (hb0, hb1), (cb0, cb1)
        hsems, csems = (sh0, sh1), (sc0, sc1)
        n_chunks = ROWS_PER_W // SC_CHUNK

        def h_copy(ch, buf, sem):
            return pltpu.make_async_copy(
                h_hbm.at[pl.ds((base + ch * SC_CHUNK) * H, SC_CHUNK * H)],
                buf, sem)

        def c_copy(ch, buf, sem):
            return pltpu.make_async_copy(
                cl_hbm.at[pl.ds(base + ch * SC_CHUNK, SC_CHUNK)], buf, sem)

        h_copy(0, hbufs[0], hsems[0]).start()
        c_copy(0, cbufs[0], csems[0]).start()
        for zc in zcopies:
            zc.wait()
        for ch in range(n_chunks):
            cur = ch % 2
            h_copy(ch, hbufs[cur], hsems[cur]).wait()
            c_copy(ch, cbufs[cur], csems[cur]).wait()
            if ch + 1 < n_chunks:
                h_copy(ch + 1, hbufs[1 - cur], hsems[1 - cur]).start()
                c_copy(ch + 1, cbufs[1 - cur], csems[1 - cur]).start()
            hb, cb = hbufs[cur], cbufs[cur]

            @pl.loop(0, SC_CHUNK, step=16)
            def _rows(t):
                cids = cb[pl.ds(t, 16)] * H
                ioti = lax.iota(I32, 16)
                for q in range(4):
                    work = []
                    for e in range(NACC):
                        j = q * NACC + e
                        cj = jnp.broadcast_to(cids[j], (16,))
                        for k in range(H // 16):
                            idx = cj + (k * 16 + ioti)
                            g = plsc.load_gather(accs[e], [idx])
                            v = hb[pl.ds((t + j) * H + k * 16, 16)]
                            work.append((e, idx, g, v))
                    for e, idx, g, v in work:
                        plsc.store_scatter(accs[e], [idx], jnp.maximum(g, v))

        for e in range(NACC):
            pltpu.sync_copy(accs[e], out_hbm.at[NACC * w + e])

    zeros = jnp.zeros((NC * H,), F32)
    return sc_kernel(h_flat, cl_flat, zeros)


def kernel(x, cluster, W0, b0, g0, beta0, W1, b1, g1, beta1, W2, b2, g2,
           beta2):
    cl_flat = cluster.astype(I32).reshape(ROWS)
    cl_g = [cl_flat[g * GROWS:(g + 1) * GROWS] for g in range(NGROUPS)]
    cl3d_g = [c.reshape(GROWS // BLK, 1, BLK) for c in cl_g]

    h0 = _layer0(x.reshape(ROWS, C), W0, b0, g0, beta0)
    hg = [h0[g * GROWS:(g + 1) * GROWS] for g in range(NGROUPS)]
    for (W, b, g_, beta) in ((W1, b1, g1, beta1), (W2, b2, g2, beta2)):
        pg = [_sc_scatter_max(hg[g].reshape(GROWS * H), cl_g[g])
              for g in range(NGROUPS)]
        tg = [_tmerge(pg[g].reshape(BG, WPB, NC, H), W[H:])
              for g in range(NGROUPS)]
        hg = [_layer_mid(hg[g], cl3d_g[g], tg[g], W, b, g_, beta)
              for g in range(NGROUPS)]
    pg = [_sc_scatter_max(hg[g].reshape(GROWS * H), cl_g[g])
          for g in range(NGROUPS)]
    p = jnp.concatenate([p_.reshape(BG, WPB, NC, H) for p_ in pg], axis=0)
    return _final(p)


# speedup vs baseline: 1.3095x; 1.3095x over previous
import dataclasses
import functools

import jax
import jax.numpy as jnp
from jax import lax
from jax.experimental import pallas as pl
from jax.experimental.pallas import tpu as pltpu
from jax.experimental.pallas import tpu_sc as plsc

B = 16
N = 4096
C = 128
H = 64
NC = 256
ROWS = B * N
BLK = 512
SC_WORKERS = 32
NGROUPS = 2
BG = B // NGROUPS
GROWS = ROWS // NGROUPS
NACC = 2
WPB = SC_WORKERS * NACC // BG
ROWS_PER_W = GROWS // SC_WORKERS
SC_CHUNK = 512
F32 = jnp.float32
I32 = jnp.int32

HIGHEST = lax.Precision.HIGHEST


def _ln_relu(h, g, beta):
    mu = jnp.mean(h, axis=-1, keepdims=True)
    var = jnp.mean((h - mu) ** 2, axis=-1, keepdims=True)
    h = (h - mu) * lax.rsqrt(var + 1e-5) * g + beta
    return jnp.maximum(h, 0.0)


def _bf16_dot(a, b):
    return lax.dot_general(a.astype(jnp.bfloat16), b.astype(jnp.bfloat16),
                           (((1,), (0,)), ((), ())),
                           preferred_element_type=F32)


def _l0_body(x_ref, w_ref, b_ref, g_ref, beta_ref, o_ref):
    h = _bf16_dot(x_ref[...], w_ref[...])
    o_ref[...] = _ln_relu(h + b_ref[...], g_ref[...], beta_ref[...])


def _layer0(x2d, W0, b0, g0, beta0, group):
    off = group * (GROWS // BLK)
    return pl.pallas_call(
        _l0_body,
        grid=(GROWS // BLK,),
        in_specs=[
            pl.BlockSpec((BLK, C), lambda i: (off + i, 0)),
            pl.BlockSpec((C, H), lambda i: (0, 0)),
            pl.BlockSpec((1, H), lambda i: (0, 0)),
            pl.BlockSpec((1, H), lambda i: (0, 0)),
            pl.BlockSpec((1, H), lambda i: (0, 0)),
        ],
        out_specs=pl.BlockSpec((BLK, H), lambda i: (i, 0)),
        out_shape=jax.ShapeDtypeStruct((GROWS, H), F32),
    )(x2d, W0, b0.reshape(1, H), g0.reshape(1, H), beta0.reshape(1, H))


def _merge_partials(p):
    m = p[0]
    for i in range(1, p.shape[0]):
        m = jnp.maximum(m, p[i])
    return m


def _lmid_body(h_ref, cl_ref, p_ref, wt_ref, wb_ref, b_ref, g_ref, beta_ref,
               o_ref):
    aggr = _merge_partials(p_ref[0])
    t = _bf16_dot(aggr, wb_ref[...])
    cl = cl_ref[0, 0, :]
    bf = jnp.bfloat16
    oh = jnp.where(
        lax.broadcasted_iota(I32, (BLK, NC), 1) == cl[:, None],
        1.0, 0.0).astype(bf)
    t_hi = t.astype(bf)
    t_lo = (t - t_hi.astype(F32)).astype(bf)
    dims = (((1,), (0,)), ((), ()))
    contrib = (lax.dot_general(oh, t_hi, dims, preferred_element_type=F32) +
               lax.dot_general(oh, t_lo, dims, preferred_element_type=F32))
    h = _bf16_dot(h_ref[...], wt_ref[...])
    h = h + contrib + b_ref[...]
    o_ref[...] = _ln_relu(h, g_ref[...], beta_ref[...])


def _layer_mid(h2d, cl3d, partials, W, b, g, beta):
    blocks_per_b = N // BLK
    return pl.pallas_call(
        _lmid_body,
        grid=(BG, blocks_per_b),
        in_specs=[
            pl.BlockSpec((BLK, H), lambda b_, i: (b_ * blocks_per_b + i, 0)),
            pl.BlockSpec((1, 1, BLK),
                         lambda b_, i: (b_ * blocks_per_b + i, 0, 0)),
            pl.BlockSpec((1, WPB, NC, H), lambda b_, i: (b_, 0, 0, 0)),
            pl.BlockSpec((H, H), lambda b_, i: (0, 0)),
            pl.BlockSpec((H, H), lambda b_, i: (0, 0)),
            pl.BlockSpec((1, H), lambda b_, i: (0, 0)),
            pl.BlockSpec((1, H), lambda b_, i: (0, 0)),
            pl.BlockSpec((1, H), lambda b_, i: (0, 0)),
        ],
        out_specs=pl.BlockSpec((BLK, H),
                               lambda b_, i: (b_ * blocks_per_b + i, 0)),
        out_shape=jax.ShapeDtypeStruct((GROWS, H), F32),
    )(h2d, cl3d, partials, W[:H], W[H:], b.reshape(1, H), g.reshape(1, H),
      beta.reshape(1, H))


def _final_body(p_ref, o_ref):
    aggr = _merge_partials(p_ref[0])
    norm = jnp.sqrt(jnp.sum(aggr * aggr, axis=0, keepdims=True))
    normed = aggr / jnp.maximum(norm, 1e-12)
    o_ref[0] = jnp.concatenate([normed, normed], axis=-1)


def _final(partials):
    return pl.pallas_call(
        _final_body,
        grid=(BG,),
        in_specs=[pl.BlockSpec((1, WPB, NC, H), lambda b_: (b_, 0, 0, 0))],
        out_specs=pl.BlockSpec((1, NC, 2 * H), lambda b_: (b_, 0, 0)),
        out_shape=jax.ShapeDtypeStruct((BG, NC, 2 * H), F32),
    )(partials)


def _sc_compiler_params():
    cp = pltpu.CompilerParams()
    if "needs_layout_passes" in pltpu.CompilerParams.__dataclass_fields__:
        cp = dataclasses.replace(cp, needs_layout_passes=False)
    return cp


def _sc_scatter_max(h_flat, cl_flat):
    mesh = plsc.VectorSubcoreMesh(core_axis_name="c", subcore_axis_name="s")

    @functools.partial(
        pl.kernel,
        out_type=jax.ShapeDtypeStruct((SC_WORKERS * NACC, NC * H), F32),
        mesh=mesh,
        compiler_params=_sc_compiler_params(),
        scratch_types=[
            [pltpu.VMEM((NC * H,), F32) for _ in range(NACC)],
            pltpu.VMEM((SC_CHUNK * H,), F32),
            pltpu.VMEM((SC_CHUNK * H,), F32),
            pltpu.VMEM((SC_CHUNK,), I32),
            pltpu.VMEM((SC_CHUNK,), I32),
            pltpu.SemaphoreType.DMA,
            pltpu.SemaphoreType.DMA,
            pltpu.SemaphoreType.DMA,
            pltpu.SemaphoreType.DMA,
        ],
    )
    def sc_kernel(h_hbm, cl_hbm, out_hbm, accs, hb0, hb1, cb0, cb1,
                  sh0, sh1, sc0, sc1):
        w = lax.axis_index("c") * 16 + lax.axis_index("s")
        base = w * ROWS_PER_W

        hbufs, cbufs = (hb0, hb1), (cb0, cb1)
        hsems, csems = (sh0, sh1), (sc0, sc1)
        n_chunks = ROWS_PER_W // SC_CHUNK

        def h_copy(ch, buf, sem):
            return pltpu.make_async_copy(
                h_hbm.at[pl.ds((base + ch * SC_CHUNK) * H, SC_CHUNK * H)],
                buf, sem)

        def c_copy(ch, buf, sem):
            return pltpu.make_async_copy(
                cl_hbm.at[pl.ds(base + ch * SC_CHUNK, SC_CHUNK)], buf, sem)

        h_copy(0, hbufs[0], hsems[0]).start()
        c_copy(0, cbufs[0], csems[0]).start()

        @pl.loop(0, NC * H, step=16)
        def _zero(i):
            z = jnp.zeros((16,), F32)
            for a in accs:
                a[pl.ds(i, 16)] = z

        for ch in range(n_chunks):
            cur = ch % 2
            h_copy(ch, hbufs[cur], hsems[cur]).wait()
            c_copy(ch, cbufs[cur], csems[cur]).wait()
            if ch + 1 < n_chunks:
                h_copy(ch + 1, hbufs[1 - cur], hsems[1 - cur]).start()
                c_copy(ch + 1, cbufs[1 - cur], csems[1 - cur]).start()
            hb, cb = hbufs[cur], cbufs[cur]

            @pl.loop(0, SC_CHUNK, step=16)
            def _rows(t):
                cids = cb[pl.ds(t, 16)] * H
                ioti = lax.iota(I32, 16)
                for q in range(16 // NACC):
                    work = []
                    for e in range(NACC):
                        j = q * NACC + e
                        cj = jnp.broadcast_to(cids[j], (16,))
                        for k in range(H // 16):
                            idx = cj + (k * 16 + ioti)
                            g = plsc.load_gather(accs[e], [idx])
                            v = hb[pl.ds((t + j) * H + k * 16, 16)]
                            work.append((e, idx, g, v))
                    for e, idx, g, v in work:
                        plsc.store_scatter(accs[e], [idx], jnp.maximum(g, v))

        for e in range(NACC):
            pltpu.sync_copy(accs[e], out_hbm.at[NACC * w + e])

    return sc_kernel(h_flat, cl_flat)


def kernel(x, cluster, W0, b0, g0, beta0, W1, b1, g1, beta1, W2, b2, g2,
           beta2):
    cl_flat = cluster.astype(I32).reshape(ROWS)
    cl_g = [cl_flat[g * GROWS:(g + 1) * GROWS] for g in range(NGROUPS)]
    cl3d_g = [c.reshape(GROWS // BLK, 1, BLK) for c in cl_g]

    x2d = x.reshape(ROWS, C)
    hg = [_layer0(x2d, W0, b0, g0, beta0, g) for g in range(NGROUPS)]
    for (W, b, g_, beta) in ((W1, b1, g1, beta1), (W2, b2, g2, beta2)):
        pg = [_sc_scatter_max(hg[g].reshape(GROWS * H), cl_g[g])
              for g in range(NGROUPS)]
        hg = [_layer_mid(hg[g], cl3d_g[g],
                         pg[g].reshape(BG, WPB, NC, H), W, b, g_, beta)
              for g in range(NGROUPS)]
    pg = [_sc_scatter_max(hg[g].reshape(GROWS * H), cl_g[g])
          for g in range(NGROUPS)]
    out_g = [_final(pg[g].reshape(BG, WPB, NC, H)) for g in range(NGROUPS)]
    return jnp.concatenate(out_g, axis=0)
